# Initial kernel scaffold; baseline (speedup 1.0000x reference)
import functools

import jax
import jax.numpy as jnp
from jax import lax
from jax.experimental import pallas as pl
from jax.experimental.pallas import tpu as pltpu
from jax.experimental.pallas import tpu_sc as plsc

SEQ = 200
DIM = 32
NC = 2
NS = 16
NW = NC * NS

CHUNK = 800
SUB = 100
NSUB = CHUNK // SUB


def _make_kernel(n_rows):
    rows_per_w = n_rows // NW
    n_chunks = rows_per_w // CHUNK
    mesh = plsc.VectorSubcoreMesh(core_axis_name="c", subcore_axis_name="s")

    @functools.partial(
        pl.kernel,
        out_type=jax.ShapeDtypeStruct((n_rows, DIM), jnp.float32),
        mesh=mesh,
        scratch_types=[
            pltpu.VMEM((SEQ, DIM), jnp.float32),
            pltpu.VMEM((NSUB, SUB), jnp.int32),
            pltpu.VMEM((CHUNK, DIM), jnp.float32),
            pltpu.SemaphoreType.DMA,
        ],
    )
    def kern(idx_hbm, table_hbm, pos_hbm, out_hbm, pos_v, idx_v, rows_v, sem):
        wid = lax.axis_index("s") * NC + lax.axis_index("c")
        pltpu.sync_copy(pos_hbm, pos_v)

        def chunk_body(c, carry):
            row_base = wid * rows_per_w + c * CHUNK
            pltpu.sync_copy(
                idx_hbm.at[pl.ds(row_base // SUB, NSUB)], idx_v)
            copies = [
                pltpu.async_copy(
                    table_hbm.at[idx_v.at[t]],
                    rows_v.at[pl.ds(t * SUB, SUB)],
                    sem,
                )
                for t in range(NSUB)
            ]
            for cp in copies:
                cp.wait()

            def pos_body(p, carry2):
                pv0 = pos_v[p, pl.ds(0, 16)]
                pv1 = pos_v[p, pl.ds(16, 16)]
                for j in range(CHUNK // SEQ):
                    r = j * SEQ + p
                    rows_v[r, pl.ds(0, 16)] = rows_v[r, pl.ds(0, 16)] + pv0
                    rows_v[r, pl.ds(16, 16)] = rows_v[r, pl.ds(16, 16)] + pv1
                return carry2

            lax.fori_loop(0, SEQ, pos_body, 0)
            pltpu.sync_copy(rows_v, out_hbm.at[pl.ds(row_base, CHUNK)])
            return carry

        lax.fori_loop(0, n_chunks, chunk_body, 0)

    return kern


def kernel(inputs, word_table, pos_table):
    batch, seq = inputs.shape
    n_rows = batch * seq
    idx = inputs.reshape(n_rows // SUB, SUB).astype(jnp.int32)
    out = _make_kernel(n_rows)(idx, word_table, pos_table)
    return out.reshape(batch, seq, DIM)


# speedup vs baseline: 1.3903x; 1.3903x over previous
import functools

import jax
import jax.numpy as jnp
from jax import lax
from jax.experimental import pallas as pl
from jax.experimental.pallas import tpu as pltpu
from jax.experimental.pallas import tpu_sc as plsc

SEQ = 200
DIM = 32
NC = 2
NS = 16
NW = NC * NS

CHUNK = 800
SUB = 100
NSUB = CHUNK // SUB


def _make_kernel(n_rows):
    rows_per_w = n_rows // NW
    n_chunks = rows_per_w // CHUNK
    mesh = plsc.VectorSubcoreMesh(core_axis_name="c", subcore_axis_name="s")

    @functools.partial(
        pl.kernel,
        out_type=jax.ShapeDtypeStruct((n_rows, DIM), jnp.float32),
        mesh=mesh,
        compiler_params=pltpu.CompilerParams(use_tc_tiling_on_sc=False),
        scratch_types=[
            pltpu.VMEM((SEQ, DIM), jnp.float32),
            pltpu.VMEM((NSUB, SUB), jnp.int32),
            pltpu.VMEM((CHUNK, DIM), jnp.float32),
            pltpu.SemaphoreType.DMA,
        ],
    )
    def kern(idx_hbm, table_hbm, pos_hbm, out_hbm, pos_v, idx_v, rows_v, sem):
        wid = lax.axis_index("s") * NC + lax.axis_index("c")
        pltpu.sync_copy(pos_hbm, pos_v)

        def chunk_body(c, carry):
            row_base = wid * rows_per_w + c * CHUNK
            pltpu.sync_copy(
                idx_hbm.at[pl.ds(pl.multiple_of(row_base // SUB, 8), NSUB)],
                idx_v)
            copies = [
                pltpu.async_copy(
                    table_hbm.at[idx_v.at[t]],
                    rows_v.at[pl.ds(t * SUB, SUB)],
                    sem,
                )
                for t in range(NSUB)
            ]
            for cp in copies:
                cp.wait()

            def pos_body(p, carry2):
                pv0 = pos_v[p, pl.ds(0, 16)]
                pv1 = pos_v[p, pl.ds(16, 16)]
                for j in range(CHUNK // SEQ):
                    r = j * SEQ + p
                    rows_v[r, pl.ds(0, 16)] = rows_v[r, pl.ds(0, 16)] + pv0
                    rows_v[r, pl.ds(16, 16)] = rows_v[r, pl.ds(16, 16)] + pv1
                return carry2

            lax.fori_loop(0, SEQ, pos_body, 0)
            pltpu.sync_copy(rows_v, out_hbm.at[pl.ds(row_base, CHUNK)])
            return carry

        lax.fori_loop(0, n_chunks, chunk_body, 0)

    return kern


def kernel(inputs, word_table, pos_table):
    batch, seq = inputs.shape
    n_rows = batch * seq
    idx = inputs.reshape(n_rows // SUB, SUB).astype(jnp.int32)
    out = _make_kernel(n_rows)(idx, word_table, pos_table)
    return out.reshape(batch, seq, DIM)


# speedup vs baseline: 1.4883x; 1.0705x over previous
import functools

import jax
import jax.numpy as jnp
from jax import lax
from jax.experimental import pallas as pl
from jax.experimental.pallas import tpu as pltpu
from jax.experimental.pallas import tpu_sc as plsc

SEQ = 200
DIM = 32
NC = 2
NS = 16
NW = NC * NS

CHUNK = 800
SUB = 100
NSUB = CHUNK // SUB


def _make_kernel(n_rows):
    rows_per_w = n_rows // NW
    n_chunks = rows_per_w // CHUNK
    idx_rows = rows_per_w // SUB
    mesh = plsc.VectorSubcoreMesh(core_axis_name="c", subcore_axis_name="s")

    @functools.partial(
        pl.kernel,
        out_type=jax.ShapeDtypeStruct((n_rows, DIM), jnp.float32),
        mesh=mesh,
        compiler_params=pltpu.CompilerParams(use_tc_tiling_on_sc=False),
        scratch_types=[
            pltpu.VMEM((SEQ, DIM), jnp.float32),
            pltpu.VMEM((idx_rows, SUB), jnp.int32),
            pltpu.VMEM((CHUNK, DIM), jnp.float32),
            pltpu.VMEM((CHUNK, DIM), jnp.float32),
            pltpu.SemaphoreType.DMA,
            pltpu.SemaphoreType.DMA,
            pltpu.SemaphoreType.DMA,
            pltpu.SemaphoreType.DMA,
        ],
    )
    def kern(idx_hbm, table_hbm, pos_hbm, out_hbm,
             pos_v, idx_v, rows_a, rows_b, gsem_a, gsem_b, ssem_a, ssem_b):
        wid = lax.axis_index("s") * NC + lax.axis_index("c")
        row0 = wid * rows_per_w
        pltpu.sync_copy(pos_hbm, pos_v)
        pltpu.sync_copy(
            idx_hbm.at[pl.ds(pl.multiple_of(row0 // SUB, 8), idx_rows)],
            idx_v)

        def fire_gather(c, rows, gsem):
            for t in range(NSUB):
                pltpu.async_copy(
                    table_hbm.at[idx_v.at[c * NSUB + t]],
                    rows.at[pl.ds(t * SUB, SUB)],
                    gsem)

        def wait_gather(rows, gsem):
            pltpu.make_async_copy(
                out_hbm.at[pl.ds(0, CHUNK)], rows, gsem).wait()

        def add_pos(rows):
            def pos_body(p, carry):
                pv0 = pos_v[p, pl.ds(0, 16)]
                pv1 = pos_v[p, pl.ds(16, 16)]
                for j in range(CHUNK // SEQ):
                    r = j * SEQ + p
                    rows[r, pl.ds(0, 16)] = rows[r, pl.ds(0, 16)] + pv0
                    rows[r, pl.ds(16, 16)] = rows[r, pl.ds(16, 16)] + pv1
                return carry
            lax.fori_loop(0, SEQ, pos_body, 0)

        def fire_store(c, rows, ssem):
            pltpu.async_copy(
                rows, out_hbm.at[pl.ds(row0 + c * CHUNK, CHUNK)], ssem)

        def wait_store(rows, ssem):
            pltpu.make_async_copy(
                rows, out_hbm.at[pl.ds(0, CHUNK)], ssem).wait()

        fire_gather(0, rows_a, gsem_a)

        def body(i2, carry):
            c0 = i2 * 2

            @pl.when(i2 > 0)
            def _():
                wait_store(rows_b, ssem_b)
            fire_gather(c0 + 1, rows_b, gsem_b)
            wait_gather(rows_a, gsem_a)
            add_pos(rows_a)
            fire_store(c0, rows_a, ssem_a)

            @pl.when(i2 < n_chunks // 2 - 1)
            def _():
                wait_store(rows_a, ssem_a)
                fire_gather(c0 + 2, rows_a, gsem_a)
            wait_gather(rows_b, gsem_b)
            add_pos(rows_b)
            fire_store(c0 + 1, rows_b, ssem_b)
            return carry

        lax.fori_loop(0, n_chunks // 2, body, 0)
        wait_store(rows_a, ssem_a)
        wait_store(rows_b, ssem_b)

    return kern


def kernel(inputs, word_table, pos_table):
    batch, seq = inputs.shape
    n_rows = batch * seq
    idx = inputs.reshape(n_rows // SUB, SUB).astype(jnp.int32)
    out = _make_kernel(n_rows)(idx, word_table, pos_table)
    return out.reshape(batch, seq, DIM)


# speedup vs baseline: 1.4907x; 1.0016x over previous
import functools

import jax
import jax.numpy as jnp
from jax import lax
from jax.experimental import pallas as pl
from jax.experimental.pallas import tpu as pltpu
from jax.experimental.pallas import tpu_sc as plsc

SEQ = 200
DIM = 32
NC = 2
NS = 16
NW = NC * NS

CHUNK = 800
SUB = 800
NSUB = CHUNK // SUB


def _make_kernel(n_rows):
    rows_per_w = n_rows // NW
    n_chunks = rows_per_w // CHUNK
    idx_rows = rows_per_w // SUB
    mesh = plsc.VectorSubcoreMesh(core_axis_name="c", subcore_axis_name="s")

    @functools.partial(
        pl.kernel,
        out_type=jax.ShapeDtypeStruct((n_rows, DIM), jnp.float32),
        mesh=mesh,
        compiler_params=pltpu.CompilerParams(use_tc_tiling_on_sc=False),
        scratch_types=[
            pltpu.VMEM((SEQ, DIM), jnp.float32),
            pltpu.VMEM((idx_rows, SUB), jnp.int32),
            pltpu.VMEM((CHUNK, DIM), jnp.float32),
            pltpu.VMEM((CHUNK, DIM), jnp.float32),
            pltpu.SemaphoreType.DMA,
            pltpu.SemaphoreType.DMA,
            pltpu.SemaphoreType.DMA,
            pltpu.SemaphoreType.DMA,
        ],
    )
    def kern(idx_hbm, table_hbm, pos_hbm, out_hbm,
             pos_v, idx_v, rows_a, rows_b, gsem_a, gsem_b, ssem_a, ssem_b):
        wid = lax.axis_index("s") * NC + lax.axis_index("c")
        row0 = wid * rows_per_w
        pltpu.sync_copy(pos_hbm, pos_v)
        pltpu.sync_copy(
            idx_hbm.at[pl.ds(pl.multiple_of(row0 // SUB, 8), idx_rows)],
            idx_v)

        def fire_gather(c, rows, gsem):
            for t in range(NSUB):
                pltpu.async_copy(
                    table_hbm.at[idx_v.at[c * NSUB + t]],
                    rows.at[pl.ds(t * SUB, SUB)],
                    gsem)

        def wait_gather(rows, gsem):
            pltpu.make_async_copy(
                out_hbm.at[pl.ds(0, CHUNK)], rows, gsem).wait()

        def add_pos(rows):
            def pos_body(p, carry):
                pv0 = pos_v[p, pl.ds(0, 16)]
                pv1 = pos_v[p, pl.ds(16, 16)]
                for j in range(CHUNK // SEQ):
                    r = j * SEQ + p
                    rows[r, pl.ds(0, 16)] = rows[r, pl.ds(0, 16)] + pv0
                    rows[r, pl.ds(16, 16)] = rows[r, pl.ds(16, 16)] + pv1
                return carry
            lax.fori_loop(0, SEQ, pos_body, 0)

        def fire_store(c, rows, ssem):
            pltpu.async_copy(
                rows, out_hbm.at[pl.ds(row0 + c * CHUNK, CHUNK)], ssem)

        def wait_store(rows, ssem):
            pltpu.make_async_copy(
                rows, out_hbm.at[pl.ds(0, CHUNK)], ssem).wait()

        fire_gather(0, rows_a, gsem_a)

        def body(i2, carry):
            c0 = i2 * 2

            @pl.when(i2 > 0)
            def _():
                wait_store(rows_b, ssem_b)
            fire_gather(c0 + 1, rows_b, gsem_b)
            wait_gather(rows_a, gsem_a)
            add_pos(rows_a)
            fire_store(c0, rows_a, ssem_a)

            @pl.when(i2 < n_chunks // 2 - 1)
            def _():
                wait_store(rows_a, ssem_a)
                fire_gather(c0 + 2, rows_a, gsem_a)
            wait_gather(rows_b, gsem_b)
            add_pos(rows_b)
            fire_store(c0 + 1, rows_b, ssem_b)
            return carry

        lax.fori_loop(0, n_chunks // 2, body, 0)
        wait_store(rows_a, ssem_a)
        wait_store(rows_b, ssem_b)

    return kern


def kernel(inputs, word_table, pos_table):
    batch, seq = inputs.shape
    n_rows = batch * seq
    idx = inputs.reshape(n_rows // SUB, SUB).astype(jnp.int32)
    out = _make_kernel(n_rows)(idx, word_table, pos_table)
    return out.reshape(batch, seq, DIM)


# speedup vs baseline: 1.4911x; 1.0003x over previous
import functools

import jax
import jax.numpy as jnp
from jax import lax
from jax.experimental import pallas as pl
from jax.experimental.pallas import tpu as pltpu
from jax.experimental.pallas import tpu_sc as plsc

SEQ = 200
DIM = 32
NC = 2
NS = 16
NW = NC * NS

BPC = 4
CHUNK = BPC * SEQ
SUB = SEQ
NSUB = CHUNK // SUB


def _make_kernel(batch, seq):
    n_rows = batch * seq
    rows_per_w = n_rows // NW
    n_chunks = rows_per_w // CHUNK
    bat_per_w = batch // NW
    mesh = plsc.VectorSubcoreMesh(core_axis_name="c", subcore_axis_name="s")

    @functools.partial(
        pl.kernel,
        out_type=jax.ShapeDtypeStruct((batch, seq, DIM), jnp.float32),
        mesh=mesh,
        compiler_params=pltpu.CompilerParams(use_tc_tiling_on_sc=False),
        scratch_types=[
            pltpu.VMEM((SEQ, DIM), jnp.float32),
            pltpu.VMEM((rows_per_w,), jnp.int32),
            pltpu.VMEM((BPC, SEQ, DIM), jnp.float32),
            pltpu.VMEM((BPC, SEQ, DIM), jnp.float32),
            pltpu.SemaphoreType.DMA,
            pltpu.SemaphoreType.DMA,
            pltpu.SemaphoreType.DMA,
            pltpu.SemaphoreType.DMA,
        ],
    )
    def kern(idx_hbm, table_hbm, pos_hbm, out_hbm,
             pos_v, idx_v, rows_a, rows_b, gsem_a, gsem_b, ssem_a, ssem_b):
        wid = lax.axis_index("s") * NC + lax.axis_index("c")
        row0 = wid * rows_per_w
        bat0 = wid * bat_per_w
        pltpu.sync_copy(pos_hbm, pos_v)
        pltpu.sync_copy(
            idx_hbm.at[pl.ds(pl.multiple_of(row0, 8), rows_per_w)], idx_v)

        def fire_gather(c, rows, gsem):
            for t in range(NSUB):
                pltpu.async_copy(
                    table_hbm.at[idx_v.at[pl.ds(c * CHUNK + t * SUB, SUB)]],
                    rows.at[t],
                    gsem)

        def wait_gather(rows, gsem):
            pltpu.make_async_copy(
                out_hbm.at[pl.ds(0, BPC)], rows, gsem).wait()

        def add_pos(rows):
            def pos_body(p, carry):
                pv0 = pos_v[p, pl.ds(0, 16)]
                pv1 = pos_v[p, pl.ds(16, 16)]
                for j in range(BPC):
                    rows[j, p, pl.ds(0, 16)] = rows[j, p, pl.ds(0, 16)] + pv0
                    rows[j, p, pl.ds(16, 16)] = (
                        rows[j, p, pl.ds(16, 16)] + pv1)
                return carry
            lax.fori_loop(0, SEQ, pos_body, 0)

        def fire_store(c, rows, ssem):
            pltpu.async_copy(
                rows, out_hbm.at[pl.ds(bat0 + c * BPC, BPC)], ssem)

        def wait_store(rows, ssem):
            pltpu.make_async_copy(
                rows, out_hbm.at[pl.ds(0, BPC)], ssem).wait()

        fire_gather(0, rows_a, gsem_a)

        def body(i2, carry):
            c0 = i2 * 2

            @pl.when(i2 > 0)
            def _():
                wait_store(rows_b, ssem_b)
            fire_gather(c0 + 1, rows_b, gsem_b)
            wait_gather(rows_a, gsem_a)
            add_pos(rows_a)
            fire_store(c0, rows_a, ssem_a)

            @pl.when(i2 < n_chunks // 2 - 1)
            def _():
                wait_store(rows_a, ssem_a)
                fire_gather(c0 + 2, rows_a, gsem_a)
            wait_gather(rows_b, gsem_b)
            add_pos(rows_b)
            fire_store(c0 + 1, rows_b, ssem_b)
            return carry

        lax.fori_loop(0, n_chunks // 2, body, 0)
        wait_store(rows_a, ssem_a)
        wait_store(rows_b, ssem_b)

    return kern


def kernel(inputs, word_table, pos_table):
    batch, seq = inputs.shape
    idx = inputs.reshape(batch * seq).astype(jnp.int32)
    return _make_kernel(batch, seq)(idx, word_table, pos_table)
